# baseline (speedup 1.0000x reference)
import functools

import jax
import jax.numpy as jnp
from jax import lax
from jax.experimental import pallas as pl
from jax.experimental.pallas import tpu as pltpu
from jax.experimental.pallas import tpu_sc as plsc


def _argmin_body(zsq_ref, esq_ref, z_ref, e_ref, out_ref, best_val):
    n = pl.program_id(0)
    m = pl.program_id(1)
    bn = e_ref.shape[0]
    bm = z_ref.shape[2]
    zt = z_ref[0]
    mm = lax.dot_general(e_ref[...].astype(jnp.bfloat16),
                         zt.astype(jnp.bfloat16),
                         (((1,), (0,)), ((), ())),
                         preferred_element_type=jnp.float32)
    dist = (esq_ref[...] + zsq_ref[...]) - mm
    rows = lax.broadcasted_iota(jnp.int32, (bn, bm), 0) + n * bn
    vmin = jnp.min(dist, axis=0, keepdims=True)
    imin = jnp.min(jnp.where(dist == vmin, rows, jnp.int32(2**30)),
                   axis=0, keepdims=True)
    vmin_r = vmin.astype(jnp.bfloat16).astype(jnp.float32)

    @pl.when(n == 0)
    def _init():
        best_val[pl.ds(m, 1), :] = vmin_r
        out_ref[pl.ds(m, 1), :] = imin

    @pl.when(n > 0)
    def _update():
        bv = best_val[pl.ds(m, 1), :]
        bi = out_ref[pl.ds(m, 1), :]
        better = vmin < bv
        newv = jnp.where(better, vmin, bv)
        best_val[pl.ds(m, 1), :] = newv.astype(jnp.bfloat16).astype(jnp.float32)
        out_ref[pl.ds(m, 1), :] = jnp.where(better, imin, bi)


def _fused_argmin(z3, embedding, zsq_row, esq_col, bm, bn):
    b, d, hw = z3.shape
    m_total = b * hw
    k_pad = embedding.shape[0]
    mt = m_total // bm
    nt = k_pad // bn
    mpb = hw // bm
    out2d = pl.pallas_call(
        _argmin_body,
        grid=(nt, mt),
        in_specs=[
            pl.BlockSpec((1, bm), lambda n, m: (0, m)),
            pl.BlockSpec((bn, 1), lambda n, m: (n, 0)),
            pl.BlockSpec((1, d, bm),
                         lambda n, m: (m // mpb, 0, m % mpb)),
            pl.BlockSpec((bn, d), lambda n, m: (n, 0)),
        ],
        out_specs=pl.BlockSpec((mt, bm), lambda n, m: (0, 0)),
        out_shape=jax.ShapeDtypeStruct((mt, bm), jnp.int32),
        scratch_shapes=[
            pltpu.VMEM((mt, bm), jnp.float32),
        ],
        compiler_params=pltpu.CompilerParams(
            dimension_semantics=("arbitrary", "arbitrary")),
    )(zsq_row, esq_col, z3, embedding)
    return out2d.reshape(m_total)


def _sc_gather_counts(embedding, idx):
    k_total, d = embedding.shape
    m_total = idx.shape[0]
    info = plsc.get_sparse_core_info()
    nc, ns = info.num_cores, info.num_subcores
    nw = nc * ns
    rows_per_w = m_total // nw
    chunk = 128
    nchunk = rows_per_w // chunk
    hrows = k_total // ns

    mesh = plsc.VectorSubcoreMesh(core_axis_name="c", subcore_axis_name="s")

    @functools.partial(
        pl.kernel, mesh=mesh,
        out_type=jax.ShapeDtypeStruct((m_total, d), jnp.float32),
        scratch_types=[
            pltpu.VMEM((chunk,), jnp.int32),
            pltpu.VMEM((chunk,), jnp.int32),
            pltpu.VMEM((chunk,), jnp.int32),
            pltpu.VMEM((chunk,), jnp.int32),
            pltpu.VMEM((chunk, d), jnp.float32),
            pltpu.VMEM((chunk, d), jnp.float32),
            pltpu.SemaphoreType.DMA,
            pltpu.SemaphoreType.DMA,
        ],
    )
    def gather_kernel(emb_hbm, idx_hbm, out_hbm,
                      idx_0, idx_1, idx_2, idx_3, rows_a, rows_b,
                      sem_a, sem_b):
        idx_v = (idx_0, idx_1, idx_2, idx_3)
        c = lax.axis_index("c")
        s = lax.axis_index("s")
        wid = s * nc + c
        base = wid * rows_per_w

        for cc in range(nchunk):
            pltpu.sync_copy(idx_hbm.at[pl.ds(base + cc * chunk, chunk)],
                            idx_v[cc])
        bufs = (rows_a, rows_b)
        sems = (sem_a, sem_b)
        for cc in range(nchunk):
            pltpu.async_copy(emb_hbm.at[idx_v[cc]], bufs[cc % 2],
                             sems[cc % 2]).wait()
            pltpu.sync_copy(bufs[cc % 2],
                            out_hbm.at[pl.ds(base + cc * chunk, chunk)])

    return gather_kernel(embedding, idx)


def kernel(z, embedding):
    b, d, h, w = z.shape
    k_total = embedding.shape[0]
    zsq = jnp.sum(jnp.transpose(z, (0, 2, 3, 1)) ** 2, axis=3)
    esq = jnp.sum(embedding ** 2, axis=1)

    bn = 2736
    k_pad = 3 * bn
    emb_pad = jnp.concatenate(
        [embedding, jnp.zeros((k_pad - k_total, d), embedding.dtype)])
    esq_pad = jnp.concatenate(
        [esq * 0.5, jnp.full((k_pad - k_total,), 1e30, esq.dtype)])
    encoding_indices = _fused_argmin(z.reshape(b, d, h * w), emb_pad,
                                     (zsq * 0.5).reshape(1, -1),
                                     esq_pad[:, None],
                                     bm=2048, bn=bn)

    quantized_flat = _sc_gather_counts(embedding, encoding_indices)
    counts = jnp.bincount(encoding_indices, length=k_total).astype(jnp.float32)

    quantized = jnp.transpose(quantized_flat.reshape(b, h, w, d), (0, 3, 1, 2))
    z_q = z + (quantized - z)
    loss = jnp.mean((z_q - z) ** 2)
    indices = encoding_indices.reshape(b, h, w)
    avg_probs = counts / (b * h * w)
    perplexity = jnp.exp(-jnp.sum(avg_probs * jnp.log(avg_probs + 1e-10)))
    used_codes = (counts > 0).astype(jnp.float32)
    return (z_q, loss, indices, perplexity, used_codes)


# speedup vs baseline: 1.1723x; 1.1723x over previous
import functools

import jax
import jax.numpy as jnp
from jax import lax
from jax.experimental import pallas as pl
from jax.experimental.pallas import tpu as pltpu
from jax.experimental.pallas import tpu_sc as plsc


def _argmin_body(zsq_ref, esq_ref, z_ref, e_ref, out_ref, best_val):
    n = pl.program_id(0)
    m = pl.program_id(1)
    bn = e_ref.shape[0]
    bm = z_ref.shape[2]
    zt = z_ref[0]
    mm = lax.dot_general(e_ref[...].astype(jnp.bfloat16),
                         zt.astype(jnp.bfloat16),
                         (((1,), (0,)), ((), ())),
                         preferred_element_type=jnp.float32)
    dist = (esq_ref[...] + zsq_ref[...]) - mm
    rows = lax.broadcasted_iota(jnp.int32, (bn, bm), 0) + n * bn
    vmin = jnp.min(dist, axis=0, keepdims=True)
    imin = jnp.min(jnp.where(dist == vmin, rows, jnp.int32(2**30)),
                   axis=0, keepdims=True)
    vmin_r = vmin.astype(jnp.bfloat16).astype(jnp.float32)

    @pl.when(n == 0)
    def _init():
        best_val[pl.ds(m, 1), :] = vmin_r
        out_ref[pl.ds(m, 1), :] = imin

    @pl.when(n > 0)
    def _update():
        bv = best_val[pl.ds(m, 1), :]
        bi = out_ref[pl.ds(m, 1), :]
        better = vmin < bv
        newv = jnp.where(better, vmin, bv)
        best_val[pl.ds(m, 1), :] = newv.astype(jnp.bfloat16).astype(jnp.float32)
        out_ref[pl.ds(m, 1), :] = jnp.where(better, imin, bi)


def _fused_argmin(z3, embedding, zsq_row, esq_col, bm, bn):
    b, d, hw = z3.shape
    m_total = b * hw
    k_pad = embedding.shape[0]
    mt = m_total // bm
    nt = k_pad // bn
    mpb = hw // bm
    out2d = pl.pallas_call(
        _argmin_body,
        grid=(nt, mt),
        in_specs=[
            pl.BlockSpec((1, bm), lambda n, m: (0, m)),
            pl.BlockSpec((bn, 1), lambda n, m: (n, 0)),
            pl.BlockSpec((1, d, bm),
                         lambda n, m: (m // mpb, 0, m % mpb)),
            pl.BlockSpec((bn, d), lambda n, m: (n, 0)),
        ],
        out_specs=pl.BlockSpec((mt, bm), lambda n, m: (0, 0)),
        out_shape=jax.ShapeDtypeStruct((mt, bm), jnp.int32),
        scratch_shapes=[
            pltpu.VMEM((mt, bm), jnp.float32),
        ],
        compiler_params=pltpu.CompilerParams(
            dimension_semantics=("arbitrary", "arbitrary")),
    )(zsq_row, esq_col, z3, embedding)
    return out2d.reshape(m_total)


def _sc_gather_counts(embedding, idx):
    k_total, d = embedding.shape
    m_total = idx.shape[0]
    info = plsc.get_sparse_core_info()
    nc, ns = info.num_cores, info.num_subcores
    nw = nc * ns
    rows_per_w = m_total // nw
    chunk = 128
    nchunk = rows_per_w // chunk
    hrows = k_total // ns

    mesh = plsc.VectorSubcoreMesh(core_axis_name="c", subcore_axis_name="s")

    @functools.partial(
        pl.kernel, mesh=mesh,
        out_type=jax.ShapeDtypeStruct((m_total, d), jnp.float32),
        scratch_types=[
            pltpu.VMEM((chunk,), jnp.int32),
            pltpu.VMEM((chunk,), jnp.int32),
            pltpu.VMEM((chunk,), jnp.int32),
            pltpu.VMEM((chunk,), jnp.int32),
            pltpu.VMEM((chunk, d), jnp.float32),
            pltpu.VMEM((chunk, d), jnp.float32),
            pltpu.SemaphoreType.DMA,
            pltpu.SemaphoreType.DMA,
        ],
    )
    def gather_kernel(emb_hbm, idx_hbm, out_hbm,
                      idx_0, idx_1, idx_2, idx_3, rows_a, rows_b,
                      sem_a, sem_b):
        idx_v = (idx_0, idx_1, idx_2, idx_3)
        c = lax.axis_index("c")
        s = lax.axis_index("s")
        wid = s * nc + c
        base = wid * rows_per_w

        for cc in range(nchunk):
            pltpu.sync_copy(idx_hbm.at[pl.ds(base + cc * chunk, chunk)],
                            idx_v[cc])
        bufs = (rows_a, rows_b)
        sems = (sem_a, sem_b)
        cps = [None, None]
        cps[0] = pltpu.async_copy(emb_hbm.at[idx_v[0]], bufs[0], sems[0])
        for cc in range(nchunk):
            if cc + 1 < nchunk:
                cps[(cc + 1) % 2] = pltpu.async_copy(
                    emb_hbm.at[idx_v[cc + 1]], bufs[(cc + 1) % 2],
                    sems[(cc + 1) % 2])
            cps[cc % 2].wait()
            pltpu.sync_copy(bufs[cc % 2],
                            out_hbm.at[pl.ds(base + cc * chunk, chunk)])

    return gather_kernel(embedding, idx)


def kernel(z, embedding):
    b, d, h, w = z.shape
    k_total = embedding.shape[0]
    zsq = jnp.sum(jnp.transpose(z, (0, 2, 3, 1)) ** 2, axis=3)
    esq = jnp.sum(embedding ** 2, axis=1)

    bn = 2736
    k_pad = 3 * bn
    emb_pad = jnp.concatenate(
        [embedding, jnp.zeros((k_pad - k_total, d), embedding.dtype)])
    esq_pad = jnp.concatenate(
        [esq * 0.5, jnp.full((k_pad - k_total,), 1e30, esq.dtype)])
    encoding_indices = _fused_argmin(z.reshape(b, d, h * w), emb_pad,
                                     (zsq * 0.5).reshape(1, -1),
                                     esq_pad[:, None],
                                     bm=1024, bn=bn)

    quantized_flat = _sc_gather_counts(embedding, encoding_indices)
    counts = jnp.bincount(encoding_indices, length=k_total).astype(jnp.float32)

    quantized = jnp.transpose(quantized_flat.reshape(b, h, w, d), (0, 3, 1, 2))
    z_q = z + (quantized - z)
    loss = jnp.mean((z_q - z) ** 2)
    indices = encoding_indices.reshape(b, h, w)
    avg_probs = counts / (b * h * w)
    perplexity = jnp.exp(-jnp.sum(avg_probs * jnp.log(avg_probs + 1e-10)))
    used_codes = (counts > 0).astype(jnp.float32)
    return (z_q, loss, indices, perplexity, used_codes)


# speedup vs baseline: 1.1904x; 1.0154x over previous
import functools

import jax
import jax.numpy as jnp
from jax import lax
from jax.experimental import pallas as pl
from jax.experimental.pallas import tpu as pltpu
from jax.experimental.pallas import tpu_sc as plsc


def _argmin_body(zsq_ref, esq_ref, z_ref, e_ref, out_ref, best_val):
    n = pl.program_id(0)
    m = pl.program_id(1)
    bn = e_ref.shape[0]
    bm = z_ref.shape[2]
    zt = z_ref[0]
    mm = lax.dot_general(e_ref[...], zt, (((1,), (0,)), ((), ())),
                         preferred_element_type=jnp.float32)
    dist = (esq_ref[...] + zsq_ref[...]) - mm
    rows = lax.broadcasted_iota(jnp.int32, (bn, bm), 0) + n * bn
    vmin = jnp.min(dist, axis=0, keepdims=True)
    imin = jnp.min(jnp.where(dist == vmin, rows, jnp.int32(2**30)),
                   axis=0, keepdims=True)
    vmin_r = vmin.astype(jnp.bfloat16).astype(jnp.float32)

    @pl.when(n == 0)
    def _init():
        best_val[pl.ds(m, 1), :] = vmin_r
        out_ref[pl.ds(m, 1), :] = imin

    @pl.when(n > 0)
    def _update():
        bv = best_val[pl.ds(m, 1), :]
        bi = out_ref[pl.ds(m, 1), :]
        better = vmin < bv
        newv = jnp.where(better, vmin, bv)
        best_val[pl.ds(m, 1), :] = newv.astype(jnp.bfloat16).astype(jnp.float32)
        out_ref[pl.ds(m, 1), :] = jnp.where(better, imin, bi)


def _fused_argmin(z3, embedding, zsq_row, esq_col, bm, bn):
    b, d, hw = z3.shape
    m_total = b * hw
    k_pad = embedding.shape[0]
    mt = m_total // bm
    nt = k_pad // bn
    mpb = hw // bm
    out2d = pl.pallas_call(
        _argmin_body,
        grid=(nt, mt),
        in_specs=[
            pl.BlockSpec((1, bm), lambda n, m: (0, m)),
            pl.BlockSpec((bn, 1), lambda n, m: (n, 0)),
            pl.BlockSpec((1, d, bm),
                         lambda n, m: (m // mpb, 0, m % mpb)),
            pl.BlockSpec((bn, d), lambda n, m: (n, 0)),
        ],
        out_specs=pl.BlockSpec((mt, bm), lambda n, m: (0, 0)),
        out_shape=jax.ShapeDtypeStruct((mt, bm), jnp.int32),
        scratch_shapes=[
            pltpu.VMEM((mt, bm), jnp.float32),
        ],
        compiler_params=pltpu.CompilerParams(
            dimension_semantics=("arbitrary", "arbitrary")),
    )(zsq_row, esq_col, z3, embedding)
    return out2d.reshape(m_total)


def _sc_gather_counts(embedding, idx):
    k_total, d = embedding.shape
    m_total = idx.shape[0]
    info = plsc.get_sparse_core_info()
    nc, ns = info.num_cores, info.num_subcores
    nw = nc * ns
    rows_per_w = m_total // nw
    chunk = 128
    nchunk = rows_per_w // chunk
    hrows = k_total // ns

    mesh = plsc.VectorSubcoreMesh(core_axis_name="c", subcore_axis_name="s")

    @functools.partial(
        pl.kernel, mesh=mesh,
        out_type=jax.ShapeDtypeStruct((m_total, d), jnp.float32),
        scratch_types=[
            pltpu.VMEM((chunk,), jnp.int32),
            pltpu.VMEM((chunk,), jnp.int32),
            pltpu.VMEM((chunk,), jnp.int32),
            pltpu.VMEM((chunk,), jnp.int32),
            pltpu.VMEM((chunk, d), jnp.float32),
            pltpu.VMEM((chunk, d), jnp.float32),
            pltpu.SemaphoreType.DMA,
            pltpu.SemaphoreType.DMA,
        ],
    )
    def gather_kernel(emb_hbm, idx_hbm, out_hbm,
                      idx_0, idx_1, idx_2, idx_3, rows_a, rows_b,
                      sem_a, sem_b):
        idx_v = (idx_0, idx_1, idx_2, idx_3)
        c = lax.axis_index("c")
        s = lax.axis_index("s")
        wid = s * nc + c
        base = wid * rows_per_w

        for cc in range(nchunk):
            pltpu.sync_copy(idx_hbm.at[pl.ds(base + cc * chunk, chunk)],
                            idx_v[cc])
        bufs = (rows_a, rows_b)
        sems = (sem_a, sem_b)
        cps = [None, None]
        cps[0] = pltpu.async_copy(emb_hbm.at[idx_v[0]], bufs[0], sems[0])
        for cc in range(nchunk):
            if cc + 1 < nchunk:
                cps[(cc + 1) % 2] = pltpu.async_copy(
                    emb_hbm.at[idx_v[cc + 1]], bufs[(cc + 1) % 2],
                    sems[(cc + 1) % 2])
            cps[cc % 2].wait()
            pltpu.sync_copy(bufs[cc % 2],
                            out_hbm.at[pl.ds(base + cc * chunk, chunk)])

    return gather_kernel(embedding, idx)


def kernel(z, embedding):
    b, d, h, w = z.shape
    k_total = embedding.shape[0]
    zsq = jnp.sum(jnp.transpose(z, (0, 2, 3, 1)) ** 2, axis=3)
    esq = jnp.sum(embedding ** 2, axis=1)

    bn = 2736
    k_pad = 3 * bn
    emb_pad = jnp.concatenate(
        [embedding.astype(jnp.bfloat16),
         jnp.zeros((k_pad - k_total, d), jnp.bfloat16)])
    esq_pad = jnp.concatenate(
        [esq * 0.5, jnp.full((k_pad - k_total,), 1e30, esq.dtype)])
    encoding_indices = _fused_argmin(z.astype(jnp.bfloat16).reshape(b, d, h * w),
                                     emb_pad,
                                     (zsq * 0.5).reshape(1, -1),
                                     esq_pad[:, None],
                                     bm=1024, bn=bn)

    quantized_flat = _sc_gather_counts(embedding, encoding_indices)
    counts = jnp.bincount(encoding_indices, length=k_total).astype(jnp.float32)

    quantized = jnp.transpose(quantized_flat.reshape(b, h, w, d), (0, 3, 1, 2))
    z_q = z + (quantized - z)
    loss = jnp.mean((z_q - z) ** 2)
    indices = encoding_indices.reshape(b, h, w)
    avg_probs = counts / (b * h * w)
    perplexity = jnp.exp(-jnp.sum(avg_probs * jnp.log(avg_probs + 1e-10)))
    used_codes = (counts > 0).astype(jnp.float32)
    return (z_q, loss, indices, perplexity, used_codes)
